# baseline (speedup 1.0000x reference)
import functools

import jax
import jax.numpy as jnp
from jax import lax
from jax.experimental import pallas as pl
from jax.experimental.pallas import tpu as pltpu
from jax.experimental.pallas import tpu_sc as plsc

B = 128
V = 100000
B_SC = 32
B_TC = B - B_SC

KS0 = 0
KS1 = 42
KS2 = KS0 ^ KS1 ^ 0x1BD11BDA
_ROTS = ((13, 15, 26, 6), (17, 29, 16, 24))
_INJECT = ((KS1, KS2, 1), (KS2, KS0, 2), (KS0, KS1, 3), (KS1, KS2, 4), (KS2, KS0, 5))

_TINY = float(jnp.finfo(jnp.float32).tiny)
_NEG_INF = float("-inf")

_LOG_POLY = (
    -3.173079160534442e-11,
    1.0000000025276106,
    -0.4999999820678256,
    0.33333278012578005,
    -0.25000127717187504,
    0.20003420797455176,
    -0.16665529578122412,
    0.14199694268775429,
    -0.12424601284408243,
    0.12017414115663498,
    -0.11631797397291235,
    0.06459239173209066,
)
_LN2 = 0.6931471805599453
_SQRT2 = 1.4142135623730951


def _threefry_bits(cnt, u32):
    if u32:
        def rotl(x, r):
            return (x << jnp.uint32(r)) | (x >> jnp.uint32(32 - r))
        cst = jnp.uint32
    else:
        def rotl(x, r):
            return (x << jnp.int32(r)) | lax.shift_right_logical(x, jnp.int32(32 - r))
        def cst(v):
            v &= 0xFFFFFFFF
            return jnp.int32(v - 0x100000000 if v >= 0x80000000 else v)
    x0 = cnt + cst(KS1)
    x1 = rotl(x0, 13) ^ x0
    first = True
    for g in range(5):
        for r in _ROTS[g % 2]:
            if first:
                first = False
                continue
            x0 = x0 + x1
            x1 = rotl(x1, r) ^ x0
        a, b, c = _INJECT[g]
        if a:
            x0 = x0 + cst(a)
        x1 = x1 + cst(b + c)
    return x0 ^ x1


R = 8
C = B_TC // R


def _tc_body(logits_ref, out_ref):
    j = pl.program_id(0)

    x = logits_ref[...]
    row = lax.broadcasted_iota(jnp.int32, (R, V), 0) + j * R
    col = lax.broadcasted_iota(jnp.int32, (R, V), 1)
    cnt = (row * V + col).astype(jnp.uint32)

    bits = _threefry_bits(cnt, u32=True)
    fl = lax.bitcast_convert_type(
        (bits >> jnp.uint32(9)) | jnp.uint32(0x3F800000), jnp.float32
    ) - jnp.float32(1.0)
    u = jnp.maximum(jnp.float32(_TINY), fl)
    g = -jnp.log(-jnp.log(u))
    vals = jnp.where(col < V, x + g, jnp.float32(_NEG_INF))

    bm = jnp.max(vals, axis=1, keepdims=True)
    bi = jnp.min(
        jnp.where(vals == bm, col, jnp.int32(0x7FFFFFFF)), axis=1, keepdims=True
    )
    out_ref[...] = jnp.broadcast_to(bi, out_ref.shape)


def _tc_sample(logits, interpret=False):
    out = pl.pallas_call(
        _tc_body,
        grid=(C,),
        in_specs=[pl.BlockSpec((R, V), lambda j: (j, 0))],
        out_specs=pl.BlockSpec((R, 128), lambda j: (j, 0)),
        out_shape=jax.ShapeDtypeStruct((B_TC, 128), jnp.int32),
        compiler_params=pltpu.CompilerParams(
            dimension_semantics=("arbitrary",),
        ),
        interpret=interpret,
    )(logits)
    return out[:, 0]


CH = 2000
NCH = V // CH
NVEC = CH // 16


def _neg_log(u):
    bx = lax.bitcast_convert_type(u, jnp.int32)
    e = lax.shift_right_logical(bx, jnp.int32(23)) - jnp.int32(127)
    m = lax.bitcast_convert_type(
        (bx & jnp.int32(0x007FFFFF)) | jnp.int32(0x3F800000), jnp.float32
    )
    big = m >= jnp.float32(_SQRT2)
    m = jnp.where(big, m * jnp.float32(0.5), m)
    ef = e.astype(jnp.float32)
    ef = jnp.where(big, ef + jnp.float32(1.0), ef)
    t = m - jnp.float32(1.0)
    acc = jnp.float32(_LOG_POLY[-1])
    for c in _LOG_POLY[-2::-1]:
        acc = acc * t + jnp.float32(c)
    return -(ef * jnp.float32(_LN2) + acc)


def _sc_body(logits_hbm, out_hbm, buf0, buf1, mbuf, ibuf, sem0, sem1):
    cix = lax.axis_index("c")
    six = lax.axis_index("s")
    w = six * 2 + cix
    rowg = w + B_TC
    base = rowg * V

    def process(buf, j, carry):
        def vec_body(v, carry):
            m, idx = carry
            l = buf[pl.ds(v * 16, 16)]
            col0 = j * CH + v * 16
            colv = lax.iota(jnp.int32, 16) + col0
            cnt = colv + base
            bits = _threefry_bits(cnt, u32=False)
            fl = lax.bitcast_convert_type(
                lax.shift_right_logical(bits, jnp.int32(9))
                | jnp.int32(0x3F800000),
                jnp.float32,
            ) - jnp.float32(1.0)
            u = jnp.maximum(jnp.float32(_TINY), fl)
            key = _neg_log(u) * jnp.exp(-l)
            better = key < m
            m = jnp.where(better, key, m)
            idx = jnp.where(better, colv, idx)
            return m, idx

        return lax.fori_loop(0, NVEC, vec_body, carry)

    def start(j, buf, sem):
        pltpu.make_async_copy(
            logits_hbm.at[pl.ds(w * V + j * CH, CH)], buf, sem
        ).start()

    def wait(buf, sem):
        pltpu.make_async_copy(logits_hbm.at[pl.ds(0, CH)], buf, sem).wait()

    start(0, buf0, sem0)
    start(1, buf1, sem1)

    def chunk_pair(k, carry):
        wait(buf0, sem0)
        carry = process(buf0, 2 * k, carry)
        start(2 * k + 2, buf0, sem0)
        wait(buf1, sem1)
        carry = process(buf1, 2 * k + 1, carry)
        start(2 * k + 3, buf1, sem1)
        return carry

    m0 = jnp.full((16,), jnp.float32(float("inf")))
    i0 = jnp.zeros((16,), jnp.int32)
    carry = lax.fori_loop(0, NCH // 2 - 1, chunk_pair, (m0, i0))
    wait(buf0, sem0)
    carry = process(buf0, NCH - 2, carry)
    wait(buf1, sem1)
    m, idx = process(buf1, NCH - 1, carry)

    mbuf[...] = lax.bitcast_convert_type(m, jnp.int32)
    ibuf[...] = idx
    pltpu.sync_copy(mbuf, out_hbm.at[pl.ds(w * 32, 16)])
    pltpu.sync_copy(ibuf, out_hbm.at[pl.ds(w * 32 + 16, 16)])


def _sc_sample(logits_bot, interpret=False):
    return pl.kernel(
        _sc_body,
        out_type=jax.ShapeDtypeStruct((B_SC * 32,), jnp.int32),
        mesh=plsc.VectorSubcoreMesh(
            core_axis_name="c", subcore_axis_name="s", num_cores=2, num_subcores=16
        ),
        scratch_types=[
            pltpu.VMEM((CH,), jnp.float32),
            pltpu.VMEM((CH,), jnp.float32),
            pltpu.VMEM((16,), jnp.int32),
            pltpu.VMEM((16,), jnp.int32),
            pltpu.SemaphoreType.DMA,
            pltpu.SemaphoreType.DMA,
        ],
        interpret=interpret,
    )(logits_bot)


@functools.partial(jax.jit, static_argnames=("interpret",))
def _sample(logits, interpret=False):
    out_sc = _sc_sample(logits[B_TC:].reshape(-1), interpret)
    out_tc = _tc_sample(logits, interpret)
    parts = out_sc.reshape(B_SC, 2, 16)
    lane = jnp.argmin(parts[:, 0, :], axis=1)
    best = jnp.take_along_axis(parts[:, 1, :], lane[:, None], axis=1)[:, 0]
    return jnp.concatenate([out_tc, best])


def kernel(logits):
    return _sample(logits).astype(jnp.int64)


# speedup vs baseline: 1.3232x; 1.3232x over previous
import functools

import jax
import jax.numpy as jnp
from jax import lax
from jax.experimental import pallas as pl
from jax.experimental.pallas import tpu as pltpu
from jax.experimental.pallas import tpu_sc as plsc

B = 128
V = 100000
B_SC = 32
B_TC = B - B_SC

KS0 = 0
KS1 = 42
KS2 = KS0 ^ KS1 ^ 0x1BD11BDA
_ROTS = ((13, 15, 26, 6), (17, 29, 16, 24))
_INJECT = ((KS1, KS2, 1), (KS2, KS0, 2), (KS0, KS1, 3), (KS1, KS2, 4), (KS2, KS0, 5))

_TINY = float(jnp.finfo(jnp.float32).tiny)
_NEG_INF = float("-inf")

_LOG_POLY = (
    -3.173079160534442e-11,
    1.0000000025276106,
    -0.4999999820678256,
    0.33333278012578005,
    -0.25000127717187504,
    0.20003420797455176,
    -0.16665529578122412,
    0.14199694268775429,
    -0.12424601284408243,
    0.12017414115663498,
    -0.11631797397291235,
    0.06459239173209066,
)
_LN2 = 0.6931471805599453
_SQRT2 = 1.4142135623730951


def _threefry_bits(cnt, u32):
    if u32:
        def rotl(x, r):
            return (x << jnp.uint32(r)) | (x >> jnp.uint32(32 - r))
        cst = jnp.uint32
    else:
        def rotl(x, r):
            return (x << jnp.int32(r)) | lax.shift_right_logical(x, jnp.int32(32 - r))
        def cst(v):
            v &= 0xFFFFFFFF
            return jnp.int32(v - 0x100000000 if v >= 0x80000000 else v)
    x0 = cnt + cst(KS1)
    x1 = rotl(x0, 13) ^ x0
    first = True
    for g in range(5):
        for r in _ROTS[g % 2]:
            if first:
                first = False
                continue
            x0 = x0 + x1
            x1 = rotl(x1, r) ^ x0
        a, b, c = _INJECT[g]
        if a:
            x0 = x0 + cst(a)
        x1 = x1 + cst(b + c)
    return x0 ^ x1


W = 2048
C = (V + W - 1) // W


def _tc_body(logits_ref, om_ref, oi_ref):
    j = pl.program_id(0)

    x = logits_ref[...]
    row = lax.broadcasted_iota(jnp.int32, (B_TC, W), 0)
    col = lax.broadcasted_iota(jnp.int32, (B_TC, W), 1) + j * W
    cnt = (row * V + col).astype(jnp.uint32)

    bits = _threefry_bits(cnt, u32=True)
    fl = lax.bitcast_convert_type(
        (bits >> jnp.uint32(9)) | jnp.uint32(0x3F800000), jnp.float32
    ) - jnp.float32(1.0)
    u = jnp.maximum(jnp.float32(_TINY), fl)
    g = -jnp.log(-jnp.log(u))
    vals = jnp.where(col < V, x + g, jnp.float32(_NEG_INF))

    bm = jnp.max(vals, axis=1, keepdims=True)
    bi = jnp.min(
        jnp.where(vals == bm, col, jnp.int32(0x7FFFFFFF)), axis=1, keepdims=True
    )
    om_ref[...] = jnp.broadcast_to(bm, om_ref.shape)
    oi_ref[...] = jnp.broadcast_to(bi, oi_ref.shape)


def _tc_sample(logits, interpret=False):
    om, oi = pl.pallas_call(
        _tc_body,
        grid=(C,),
        in_specs=[pl.BlockSpec((B_TC, W), lambda j: (0, j))],
        out_specs=[
            pl.BlockSpec((B_TC, 128), lambda j: (0, j)),
            pl.BlockSpec((B_TC, 128), lambda j: (0, j)),
        ],
        out_shape=[
            jax.ShapeDtypeStruct((B_TC, C * 128), jnp.float32),
            jax.ShapeDtypeStruct((B_TC, C * 128), jnp.int32),
        ],
        compiler_params=pltpu.CompilerParams(
            dimension_semantics=("parallel",),
        ),
        interpret=interpret,
    )(logits)
    m = om[:, ::128]
    i = oi[:, ::128]
    lane = jnp.argmax(m, axis=1)
    return jnp.take_along_axis(i, lane[:, None], axis=1)[:, 0]


CH = 2000
NCH = V // CH
NVEC = CH // 16


def _neg_log(u):
    bx = lax.bitcast_convert_type(u, jnp.int32)
    e = lax.shift_right_logical(bx, jnp.int32(23)) - jnp.int32(127)
    m = lax.bitcast_convert_type(
        (bx & jnp.int32(0x007FFFFF)) | jnp.int32(0x3F800000), jnp.float32
    )
    big = m >= jnp.float32(_SQRT2)
    m = jnp.where(big, m * jnp.float32(0.5), m)
    ef = e.astype(jnp.float32)
    ef = jnp.where(big, ef + jnp.float32(1.0), ef)
    t = m - jnp.float32(1.0)
    acc = jnp.float32(_LOG_POLY[-1])
    for c in _LOG_POLY[-2::-1]:
        acc = acc * t + jnp.float32(c)
    return -(ef * jnp.float32(_LN2) + acc)


def _sc_body(logits_hbm, out_hbm, buf0, buf1, mbuf, ibuf, sem0, sem1):
    cix = lax.axis_index("c")
    six = lax.axis_index("s")
    w = six * 2 + cix
    rowg = w + B_TC
    base = rowg * V

    def process(buf, j, carry):
        def one_vec(v, carry):
            m, idx = carry
            l = buf[pl.ds(v * 16, 16)]
            col0 = j * CH + v * 16
            colv = lax.iota(jnp.int32, 16) + col0
            cnt = colv + base
            bits = _threefry_bits(cnt, u32=False)
            fl = lax.bitcast_convert_type(
                lax.shift_right_logical(bits, jnp.int32(9))
                | jnp.int32(0x3F800000),
                jnp.float32,
            ) - jnp.float32(1.0)
            u = jnp.maximum(jnp.float32(_TINY), fl)
            key = _neg_log(u) * jnp.exp(-l)
            better = key < m
            m = jnp.where(better, key, m)
            idx = jnp.where(better, colv, idx)
            return m, idx

        def vec5(v5, carry):
            for t in range(5):
                carry = one_vec(v5 * 5 + t, carry)
            return carry

        return lax.fori_loop(0, NVEC // 5, vec5, carry)

    def start(j, buf, sem):
        pltpu.make_async_copy(
            logits_hbm.at[pl.ds(w * V + j * CH, CH)], buf, sem
        ).start()

    def wait(buf, sem):
        pltpu.make_async_copy(logits_hbm.at[pl.ds(0, CH)], buf, sem).wait()

    start(0, buf0, sem0)
    start(1, buf1, sem1)

    def chunk_pair(k, carry):
        wait(buf0, sem0)
        carry = process(buf0, 2 * k, carry)
        start(2 * k + 2, buf0, sem0)
        wait(buf1, sem1)
        carry = process(buf1, 2 * k + 1, carry)
        start(2 * k + 3, buf1, sem1)
        return carry

    m0 = jnp.full((16,), jnp.float32(float("inf")))
    i0 = jnp.zeros((16,), jnp.int32)
    carry = lax.fori_loop(0, NCH // 2 - 1, chunk_pair, (m0, i0))
    wait(buf0, sem0)
    carry = process(buf0, NCH - 2, carry)
    wait(buf1, sem1)
    m, idx = process(buf1, NCH - 1, carry)

    mbuf[...] = lax.bitcast_convert_type(m, jnp.int32)
    ibuf[...] = idx
    pltpu.sync_copy(mbuf, out_hbm.at[pl.ds(w * 32, 16)])
    pltpu.sync_copy(ibuf, out_hbm.at[pl.ds(w * 32 + 16, 16)])


def _sc_sample(logits_bot, interpret=False):
    return pl.kernel(
        _sc_body,
        out_type=jax.ShapeDtypeStruct((B_SC * 32,), jnp.int32),
        mesh=plsc.VectorSubcoreMesh(
            core_axis_name="c", subcore_axis_name="s", num_cores=2, num_subcores=16
        ),
        scratch_types=[
            pltpu.VMEM((CH,), jnp.float32),
            pltpu.VMEM((CH,), jnp.float32),
            pltpu.VMEM((16,), jnp.int32),
            pltpu.VMEM((16,), jnp.int32),
            pltpu.SemaphoreType.DMA,
            pltpu.SemaphoreType.DMA,
        ],
        interpret=interpret,
    )(logits_bot)


@functools.partial(jax.jit, static_argnames=("interpret",))
def _sample(logits, interpret=False):
    out_sc = _sc_sample(logits[B_TC:].reshape(-1), interpret)
    out_tc = _tc_sample(logits, interpret)
    parts = out_sc.reshape(B_SC, 2, 16)
    lane = jnp.argmin(parts[:, 0, :], axis=1)
    best = jnp.take_along_axis(parts[:, 1, :], lane[:, None], axis=1)[:, 0]
    return jnp.concatenate([out_tc, best])


def kernel(logits):
    return _sample(logits).astype(jnp.int64)


# speedup vs baseline: 1.5461x; 1.1684x over previous
import functools

import jax
import jax.numpy as jnp
from jax import lax
from jax.experimental import pallas as pl
from jax.experimental.pallas import tpu as pltpu
from jax.experimental.pallas import tpu_sc as plsc

B = 128
V = 100000
B_SC = 32
B_TC = B - B_SC

KS0 = 0
KS1 = 42
KS2 = KS0 ^ KS1 ^ 0x1BD11BDA
_ROTS = ((13, 15, 26, 6), (17, 29, 16, 24))
_INJECT = ((KS1, KS2, 1), (KS2, KS0, 2), (KS0, KS1, 3), (KS1, KS2, 4), (KS2, KS0, 5))

_TINY = float(jnp.finfo(jnp.float32).tiny)
_NEG_INF = float("-inf")

_LOG_POLY = (
    -3.173079160534442e-11,
    1.0000000025276106,
    -0.4999999820678256,
    0.33333278012578005,
    -0.25000127717187504,
    0.20003420797455176,
    -0.16665529578122412,
    0.14199694268775429,
    -0.12424601284408243,
    0.12017414115663498,
    -0.11631797397291235,
    0.06459239173209066,
)
_LN2 = 0.6931471805599453
_SQRT2 = 1.4142135623730951


def _threefry_bits(cnt, u32):
    if u32:
        def rotl(x, r):
            return (x << jnp.uint32(r)) | (x >> jnp.uint32(32 - r))
        cst = jnp.uint32
    else:
        def rotl(x, r):
            return (x << jnp.int32(r)) | lax.shift_right_logical(x, jnp.int32(32 - r))
        def cst(v):
            v &= 0xFFFFFFFF
            return jnp.int32(v - 0x100000000 if v >= 0x80000000 else v)
    x0 = cnt + cst(KS1)
    x1 = rotl(x0, 13) ^ x0
    first = True
    for g in range(5):
        for r in _ROTS[g % 2]:
            if first:
                first = False
                continue
            x0 = x0 + x1
            x1 = rotl(x1, r) ^ x0
        a, b, c = _INJECT[g]
        if a:
            x0 = x0 + cst(a)
        x1 = x1 + cst(b + c)
    return x0 ^ x1


W = 2048
C = (V + W - 1) // W


def _tc_body(logits_ref, om_ref, oi_ref):
    j = pl.program_id(0)

    x = logits_ref[...]
    row = lax.broadcasted_iota(jnp.int32, (B_TC, W), 0)
    col = lax.broadcasted_iota(jnp.int32, (B_TC, W), 1) + j * W
    cnt = (row * V + col).astype(jnp.uint32)

    bits = _threefry_bits(cnt, u32=True)
    fl = lax.bitcast_convert_type(
        (bits >> jnp.uint32(9)) | jnp.uint32(0x3F800000), jnp.float32
    ) - jnp.float32(1.0)
    u = jnp.maximum(jnp.float32(_TINY), fl)
    g = -jnp.log(-jnp.log(u))
    vals = jnp.where(col < V, x + g, jnp.float32(_NEG_INF))

    bm = jnp.max(vals, axis=1, keepdims=True)
    bi = jnp.min(
        jnp.where(vals == bm, col, jnp.int32(0x7FFFFFFF)), axis=1, keepdims=True
    )
    om_ref[...] = jnp.broadcast_to(bm, om_ref.shape)
    oi_ref[...] = jnp.broadcast_to(bi, oi_ref.shape)


def _tc_sample(logits, interpret=False):
    om, oi = pl.pallas_call(
        _tc_body,
        grid=(C,),
        in_specs=[pl.BlockSpec((B_TC, W), lambda j: (0, j))],
        out_specs=[
            pl.BlockSpec((B_TC, 128), lambda j: (0, j)),
            pl.BlockSpec((B_TC, 128), lambda j: (0, j)),
        ],
        out_shape=[
            jax.ShapeDtypeStruct((B_TC, C * 128), jnp.float32),
            jax.ShapeDtypeStruct((B_TC, C * 128), jnp.int32),
        ],
        compiler_params=pltpu.CompilerParams(
            dimension_semantics=("parallel",),
        ),
        interpret=interpret,
    )(logits)
    m = om[:, ::128]
    i = oi[:, ::128]
    lane = jnp.argmax(m, axis=1)
    return jnp.take_along_axis(i, lane[:, None], axis=1)[:, 0]


CH = 2000
NCH = V // CH
NVEC = CH // 16


def _neg_log(u):
    bx = lax.bitcast_convert_type(u, jnp.int32)
    e = lax.shift_right_logical(bx, jnp.int32(23)) - jnp.int32(127)
    m = lax.bitcast_convert_type(
        (bx & jnp.int32(0x007FFFFF)) | jnp.int32(0x3F800000), jnp.float32
    )
    big = m >= jnp.float32(_SQRT2)
    m = jnp.where(big, m * jnp.float32(0.5), m)
    ef = e.astype(jnp.float32)
    ef = jnp.where(big, ef + jnp.float32(1.0), ef)
    t = m - jnp.float32(1.0)
    acc = jnp.float32(_LOG_POLY[-1])
    for c in _LOG_POLY[-2::-1]:
        acc = acc * t + jnp.float32(c)
    return -(ef * jnp.float32(_LN2) + acc)


GCH = 2048
NFULL = V // GCH
NU = NFULL // 8
STRIP0 = NFULL * GCH
STRIPC = V - STRIP0
STRIPV = STRIPC // 16


def _sc_body(logits_hbm, out_hbm, buf0, buf1, sbuf, obuf, sem0, sem1):
    cix = lax.axis_index("c")
    six = lax.axis_index("s")
    w = six * 2 + cix
    a = w & 3
    s = lax.shift_right_logical(w, 2)
    row0 = B_TC + 8 * a

    def one_vec(buf, r, base, col0, v, mr, ir, masked):
        l = buf[r, pl.ds(v * 16, 16)]
        colv = lax.iota(jnp.int32, 16) + (col0 + v * 16)
        cnt = colv + base
        bits = _threefry_bits(cnt, u32=False)
        fl = lax.bitcast_convert_type(
            lax.shift_right_logical(bits, jnp.int32(9)) | jnp.int32(0x3F800000),
            jnp.float32,
        ) - jnp.float32(1.0)
        u = jnp.maximum(jnp.float32(_TINY), fl)
        key = _neg_log(u) * jnp.exp(-l)
        if masked:
            key = jnp.where(colv < jnp.int32(V), key, jnp.float32(float("inf")))
        better = key < mr
        mr = jnp.where(better, key, mr)
        ir = jnp.where(better, colv, ir)
        return mr, ir

    def process(buf, col0, carry):
        out = []
        for r in range(8):
            base = (row0 + r) * V

            def vec1(v, c, r=r, base=base):
                mr, ir = c
                return one_vec(buf, r, base, col0, v, mr, ir, False)

            out.append(lax.fori_loop(0, GCH // 16, vec1, carry[r]))
        return tuple(out)

    def cb(u):
        return (s + 8 * u) * GCH

    def start(u, buf, sem):
        uc = jnp.minimum(u, jnp.int32(NU - 1))
        pltpu.make_async_copy(
            logits_hbm.at[
                pl.ds(pl.multiple_of(row0, 8), 8),
                pl.ds(pl.multiple_of(cb(uc), 128), GCH),
            ],
            buf,
            sem,
        ).start()

    def wait(buf, sem):
        pltpu.make_async_copy(
            logits_hbm.at[pl.ds(0, 8), pl.ds(0, GCH)], buf, sem
        ).wait()

    start(0, buf0, sem0)
    start(1, buf1, sem1)

    def chunk_pair(k, carry):
        wait(buf0, sem0)
        carry = process(buf0, cb(2 * k), carry)
        start(2 * k + 2, buf0, sem0)
        wait(buf1, sem1)
        carry = process(buf1, cb(2 * k + 1), carry)
        start(2 * k + 3, buf1, sem1)
        return carry

    m0 = jnp.full((16,), jnp.float32(float("inf")))
    i0 = jnp.zeros((16,), jnp.int32)
    carry = tuple((m0, i0) for _ in range(8))
    carry = lax.fori_loop(0, NU // 2, chunk_pair, carry)
    wait(buf0, sem0)
    wait(buf1, sem1)

    pltpu.sync_copy(
        logits_hbm.at[pl.ds(pl.multiple_of(row0, 8), 8), pl.ds(STRIP0, STRIPC)],
        sbuf,
    )
    out = []
    for r in range(8):
        base = (row0 + r) * V

        def strip_vec(t, c, r=r, base=base):
            mr, ir = c
            v = jnp.minimum(s + 8 * t, jnp.int32(STRIPV - 1))
            return one_vec(sbuf, r, base, STRIP0, v, mr, ir, True)

        out.append(lax.fori_loop(0, 14, strip_vec, carry[r]))

    for r in range(8):
        mr, ir = out[r]
        obuf[r, pl.ds(0, 16)] = lax.bitcast_convert_type(mr, jnp.int32)
        obuf[r, pl.ds(16, 16)] = ir
    pltpu.sync_copy(obuf, out_hbm.at[pl.ds(pl.multiple_of(w * 8, 8), 8), :])


def _sc_sample(logits, interpret=False):
    return pl.kernel(
        _sc_body,
        out_type=jax.ShapeDtypeStruct((32 * 8, 32), jnp.int32),
        mesh=plsc.VectorSubcoreMesh(
            core_axis_name="c", subcore_axis_name="s", num_cores=2, num_subcores=16
        ),
        scratch_types=[
            pltpu.VMEM((8, GCH), jnp.float32),
            pltpu.VMEM((8, GCH), jnp.float32),
            pltpu.VMEM((8, STRIPC), jnp.float32),
            pltpu.VMEM((8, 32), jnp.int32),
            pltpu.SemaphoreType.DMA,
            pltpu.SemaphoreType.DMA,
        ],
        interpret=interpret,
    )(logits)


@functools.partial(jax.jit, static_argnames=("interpret",))
def _sample(logits, interpret=False):
    out_sc = _sc_sample(logits, interpret)
    out_tc = _tc_sample(logits, interpret)
    parts = out_sc.reshape(8, 4, 8, 2, 16)
    keys = parts[:, :, :, 0, :].transpose(1, 2, 0, 3).reshape(B_SC, 128)
    idxs = parts[:, :, :, 1, :].transpose(1, 2, 0, 3).reshape(B_SC, 128)
    lane = jnp.argmin(keys, axis=1)
    best = jnp.take_along_axis(idxs, lane[:, None], axis=1)[:, 0]
    return jnp.concatenate([out_tc, best])


def kernel(logits):
    return _sample(logits).astype(jnp.int64)
